# baseline (speedup 1.0000x reference)
import jax
import jax.numpy as jnp
from jax import lax
from jax.experimental import pallas as pl
from jax.experimental.pallas import tpu as pltpu
from jax.experimental.pallas import tpu_sc as plsc

N_ATOMS = 50000
N_EDGES = 1600000
NUM_CORES = 2
NUM_SUBCORES = 16
NW = NUM_CORES * NUM_SUBCORES
NCHUNK = N_EDGES // 128
CPW = 390
NTAIL = NCHUNK - NW * CPW
BLKC = 15
BLKE = BLKC * 128
NBLK = CPW // BLKC
APW = 1568


def _build_body(ax_hbm, ay_hbm, az_hbm, t8_hbm, cx, cy, cz, rows8, sem):
    wid = lax.axis_index("s") * NUM_CORES + lax.axis_index("c")
    iota16 = lax.iota(jnp.int32, 16)
    col0 = iota16 * 0
    col1 = col0 + 1
    col2 = col0 + 2

    a0 = wid * APW
    n = jnp.minimum(APW, N_ATOMS - a0)
    pltpu.sync_copy(ax_hbm.at[pl.ds(a0, APW - 176)], cx.at[pl.ds(0, APW - 176)])
    pltpu.sync_copy(ay_hbm.at[pl.ds(a0, APW - 176)], cy.at[pl.ds(0, APW - 176)])
    pltpu.sync_copy(az_hbm.at[pl.ds(a0, APW - 176)], cz.at[pl.ds(0, APW - 176)])

    @pl.when(n > APW - 176)
    def _():
        pltpu.sync_copy(ax_hbm.at[pl.ds(a0 + APW - 176, 176)],
                        cx.at[pl.ds(APW - 176, 176)])
        pltpu.sync_copy(ay_hbm.at[pl.ds(a0 + APW - 176, 176)],
                        cy.at[pl.ds(APW - 176, 176)])
        pltpu.sync_copy(az_hbm.at[pl.ds(a0 + APW - 176, 176)],
                        cz.at[pl.ds(APW - 176, 176)])

    def grp(g, carry):
        o = g * 16
        rows = iota16 + o
        plsc.store_scatter(rows8, [rows, col0], cx[pl.ds(o, 16)])
        plsc.store_scatter(rows8, [rows, col1], cy[pl.ds(o, 16)])
        plsc.store_scatter(rows8, [rows, col2], cz[pl.ds(o, 16)])
        return carry

    lax.fori_loop(0, APW // 16, grp, 0, unroll=False)
    pltpu.sync_copy(rows8.at[pl.ds(0, APW - 176)],
                    t8_hbm.at[pl.ds(a0, APW - 176)])

    @pl.when(n > APW - 176)
    def _():
        pltpu.sync_copy(rows8.at[pl.ds(APW - 176, 176)],
                        t8_hbm.at[pl.ds(a0 + APW - 176, 176)])


def _sc_body(t8_hbm, bi_hbm, bj_hbm, ovec_hbm, odist_hbm,
             ii0, ij0, ii1, ij1,
             g0i, g0j, g1i, g1j,
             o0v, o0d, o1v, o1d,
             sg0, sg1, si0, si1, so0, so1):
    wid = lax.axis_index("s") * NUM_CORES + lax.axis_index("c")
    q_base = wid * CPW
    iota16 = lax.iota(jnp.int32, 16)
    col0 = iota16 * 0
    col1 = col0 + 1
    col2 = col0 + 2

    idx = [(ii0, ij0), (ii1, ij1)]
    gath = [(g0i, g0j), (g1i, g1j)]
    outs = [(o0v, o0d), (o1v, o1d)]
    sgs = [sg0, sg1]
    sis = [si0, si1]
    sos = [so0, so1]

    def issue_idx(b):
        p = b % 2
        e0 = (q_base + b * BLKC) * 128
        return (pltpu.async_copy(bi_hbm.at[pl.ds(e0, BLKE)], idx[p][0], sis[p]),
                pltpu.async_copy(bj_hbm.at[pl.ds(e0, BLKE)], idx[p][1], sis[p]))

    def issue_gathers(b):
        p = b % 2
        gi, gj = idx[p]
        ri, rj = gath[p]
        s = sgs[p]
        return (pltpu.async_copy(t8_hbm.at[gi], ri, s),
                pltpu.async_copy(t8_hbm.at[gj], rj, s))

    def issue_outs(b):
        p = b % 2
        vec_v, dist_v = outs[p]
        q0 = q_base + b * BLKC
        s = sos[p]
        return (pltpu.async_copy(vec_v, ovec_hbm.at[pl.ds(q0 * 512, BLKC * 512)], s),
                pltpu.async_copy(dist_v, odist_hbm.at[pl.ds(q0 * 128, BLKE)], s))

    def compute(b):
        p = b % 2
        ri, rj = gath[p]
        vec_v, dist_v = outs[p]

        def group_body(g, carry):
            o = g * 16
            rows = iota16 + o
            v0 = plsc.load_gather(rj, [rows, col0]) - plsc.load_gather(ri, [rows, col0])
            v1 = plsc.load_gather(rj, [rows, col1]) - plsc.load_gather(ri, [rows, col1])
            v2 = plsc.load_gather(rj, [rows, col2]) - plsc.load_gather(ri, [rows, col2])
            vb = (g >> 3) * 512 + (g & 7) * 16
            vec_v[pl.ds(vb, 16)] = v0
            vec_v[pl.ds(vb + 128, 16)] = v1
            vec_v[pl.ds(vb + 256, 16)] = v2
            d2 = v0 * v0 + v1 * v1 + v2 * v2
            d2c = jnp.maximum(d2, 1.1754944e-38)
            y = plsc.bitcast(0x5F3759DF - (plsc.bitcast(d2c, jnp.int32) >> 1),
                             jnp.float32)
            y = y * (1.5 - 0.5 * d2c * y * y)
            y = y * (1.5 - 0.5 * d2c * y * y)
            y = y * (1.5 - 0.5 * d2c * y * y)
            dist_v[pl.ds(o, 16)] = d2 * y
            return carry

        lax.fori_loop(0, BLKE // 16, group_body, 0, unroll=False)

    idx_descs = {0: issue_idx(0)}
    for d in idx_descs[0]:
        d.wait()
    g_descs = {0: issue_gathers(0)}
    idx_descs[1] = issue_idx(1)
    out_descs = {}
    for b in range(NBLK):
        if b + 1 < NBLK:
            for d in idx_descs[b + 1]:
                d.wait()
            g_descs[b + 1] = issue_gathers(b + 1)
        for d in g_descs[b]:
            d.wait()
        if b + 2 < NBLK:
            idx_descs[b + 2] = issue_idx(b + 2)
        if b - 2 in out_descs:
            for d in out_descs[b - 2]:
                d.wait()
        compute(b)
        out_descs[b] = issue_outs(b)
    for d in out_descs[NBLK - 2]:
        d.wait()
    for d in out_descs[NBLK - 1]:
        d.wait()

    @pl.when(wid < NTAIL)
    def _():
        qt = NW * CPW + wid
        et = qt * 128
        gi = idx[0][0].at[pl.ds(0, 128)]
        gj = idx[0][1].at[pl.ds(0, 128)]
        pltpu.sync_copy(bi_hbm.at[pl.ds(et, 128)], gi)
        pltpu.sync_copy(bj_hbm.at[pl.ds(et, 128)], gj)
        ri = gath[0][0].at[pl.ds(0, 128)]
        rj = gath[0][1].at[pl.ds(0, 128)]
        c1 = pltpu.async_copy(t8_hbm.at[gi], ri, sg0)
        c2 = pltpu.async_copy(t8_hbm.at[gj], rj, sg0)
        c1.wait()
        c2.wait()
        vec_v, dist_v = outs[0]

        def tail_group(g, carry):
            o = g * 16
            rows = iota16 + o
            u0 = plsc.load_gather(rj, [rows, col0]) - plsc.load_gather(ri, [rows, col0])
            u1 = plsc.load_gather(rj, [rows, col1]) - plsc.load_gather(ri, [rows, col1])
            u2 = plsc.load_gather(rj, [rows, col2]) - plsc.load_gather(ri, [rows, col2])
            vec_v[pl.ds(o, 16)] = u0
            vec_v[pl.ds(o + 128, 16)] = u1
            vec_v[pl.ds(o + 256, 16)] = u2
            d2 = u0 * u0 + u1 * u1 + u2 * u2
            d2c = jnp.maximum(d2, 1.1754944e-38)
            y = plsc.bitcast(0x5F3759DF - (plsc.bitcast(d2c, jnp.int32) >> 1),
                             jnp.float32)
            y = y * (1.5 - 0.5 * d2c * y * y)
            y = y * (1.5 - 0.5 * d2c * y * y)
            y = y * (1.5 - 0.5 * d2c * y * y)
            dist_v[pl.ds(o, 16)] = d2 * y
            return carry

        lax.fori_loop(0, 8, tail_group, 0, unroll=False)
        pltpu.sync_copy(vec_v.at[pl.ds(0, 512)],
                        ovec_hbm.at[pl.ds(qt * 512, 512)])
        pltpu.sync_copy(dist_v.at[pl.ds(0, 128)],
                        odist_hbm.at[pl.ds(et, 128)])


@jax.jit
def _sc_call(ax, ay, az, bi, bj):
    mesh = plsc.VectorSubcoreMesh(core_axis_name="c", subcore_axis_name="s",
                                  num_cores=NUM_CORES,
                                  num_subcores=NUM_SUBCORES)
    cparams = pltpu.CompilerParams(needs_layout_passes=False,
                                   use_tc_tiling_on_sc=False)
    build = pl.kernel(
        _build_body,
        out_type=jax.ShapeDtypeStruct((N_ATOMS, 8), jnp.float32),
        mesh=mesh,
        compiler_params=cparams,
        scratch_types=[
            pltpu.VMEM((APW,), jnp.float32),
            pltpu.VMEM((APW,), jnp.float32),
            pltpu.VMEM((APW,), jnp.float32),
            pltpu.VMEM((APW, 8), jnp.float32),
            pltpu.SemaphoreType.DMA,
        ],
    )
    t8 = build(ax, ay, az)
    ivec = pltpu.VMEM((BLKE,), jnp.int32)
    rvec = pltpu.VMEM((BLKE, 8), jnp.float32)
    vvec = pltpu.VMEM((BLKC * 512,), jnp.float32)
    dvec = pltpu.VMEM((BLKE,), jnp.float32)
    f = pl.kernel(
        _sc_body,
        out_type=(jax.ShapeDtypeStruct((NCHUNK * 512,), jnp.float32),
                  jax.ShapeDtypeStruct((N_EDGES,), jnp.float32)),
        mesh=mesh,
        compiler_params=cparams,
        scratch_types=(
            [ivec] * 4 + [rvec] * 4 + [vvec, dvec, vvec, dvec]
            + [pltpu.SemaphoreType.DMA] * 6
        ),
    )
    return f(t8, bi, bj)


def kernel(atoms_x, bonds_i, bonds_j):
    vec_raw, dist = _sc_call(atoms_x[:, 0], atoms_x[:, 1], atoms_x[:, 2],
                             bonds_i.astype(jnp.int32),
                             bonds_j.astype(jnp.int32))
    bond_vec = (vec_raw.reshape(NCHUNK, 4, 128)
                .transpose(0, 2, 1)
                .reshape(N_EDGES, 4)[:, :3])
    return bond_vec, dist


# speedup vs baseline: 1.0164x; 1.0164x over previous
import jax
import jax.numpy as jnp
from jax import lax
from jax.experimental import pallas as pl
from jax.experimental.pallas import tpu as pltpu
from jax.experimental.pallas import tpu_sc as plsc

N_ATOMS = 50000
N_EDGES = 1600000
NUM_CORES = 2
NUM_SUBCORES = 16
NW = NUM_CORES * NUM_SUBCORES
NCHUNK = N_EDGES // 128
CPW = 390
NTAIL = NCHUNK - NW * CPW
BLKC = 15
BLKE = BLKC * 128
NBLK = CPW // BLKC
APW = N_ATOMS // 25


def _build_body(ax_hbm, ay_hbm, az_hbm, t8_hbm, cx, cy, cz, rows8, sem):
    wid = lax.axis_index("s") * NUM_CORES + lax.axis_index("c")
    iota16 = lax.iota(jnp.int32, 16)
    col0 = iota16 * 0
    col1 = col0 + 1
    col2 = col0 + 2

    @pl.when(wid < 25)
    def _():
        a0 = wid * APW
        pltpu.sync_copy(ax_hbm.at[pl.ds(a0, APW)], cx)
        pltpu.sync_copy(ay_hbm.at[pl.ds(a0, APW)], cy)
        pltpu.sync_copy(az_hbm.at[pl.ds(a0, APW)], cz)

        def grp(g, carry):
            o = g * 16
            rows = iota16 + o
            plsc.store_scatter(rows8, [rows, col0], cx[pl.ds(o, 16)])
            plsc.store_scatter(rows8, [rows, col1], cy[pl.ds(o, 16)])
            plsc.store_scatter(rows8, [rows, col2], cz[pl.ds(o, 16)])
            return carry

        lax.fori_loop(0, APW // 16, grp, 0, unroll=False)
        pltpu.sync_copy(rows8, t8_hbm.at[pl.ds(a0, APW)])


def _sc_body(t8_hbm, bi_hbm, bj_hbm, ovec_hbm, odist_hbm,
             ii0, ij0, ii1, ij1,
             g0i, g0j, g1i, g1j,
             o0v, o0d, o1v, o1d,
             sg0, sg1, si0, si1, so0, so1):
    wid = lax.axis_index("s") * NUM_CORES + lax.axis_index("c")
    q_base = wid * CPW
    iota16 = lax.iota(jnp.int32, 16)
    col0 = iota16 * 0
    col1 = col0 + 1
    col2 = col0 + 2

    idx = [(ii0, ij0), (ii1, ij1)]
    gath = [(g0i, g0j), (g1i, g1j)]
    outs = [(o0v, o0d), (o1v, o1d)]
    sgs = [sg0, sg1]
    sis = [si0, si1]
    sos = [so0, so1]

    def issue_idx(b):
        p = b % 2
        e0 = (q_base + b * BLKC) * 128
        return (pltpu.async_copy(bi_hbm.at[pl.ds(e0, BLKE)], idx[p][0], sis[p]),
                pltpu.async_copy(bj_hbm.at[pl.ds(e0, BLKE)], idx[p][1], sis[p]))

    def issue_gathers(b):
        p = b % 2
        gi, gj = idx[p]
        ri, rj = gath[p]
        s = sgs[p]
        return (pltpu.async_copy(t8_hbm.at[gi], ri, s),
                pltpu.async_copy(t8_hbm.at[gj], rj, s))

    def issue_outs(b):
        p = b % 2
        vec_v, dist_v = outs[p]
        q0 = q_base + b * BLKC
        s = sos[p]
        return (pltpu.async_copy(vec_v, ovec_hbm.at[pl.ds(q0 * 512, BLKC * 512)], s),
                pltpu.async_copy(dist_v, odist_hbm.at[pl.ds(q0 * 128, BLKE)], s))

    def compute(b):
        p = b % 2
        ri, rj = gath[p]
        vec_v, dist_v = outs[p]

        def group_body(g, carry):
            o = g * 16
            rows = iota16 + o
            v0 = plsc.load_gather(rj, [rows, col0]) - plsc.load_gather(ri, [rows, col0])
            v1 = plsc.load_gather(rj, [rows, col1]) - plsc.load_gather(ri, [rows, col1])
            v2 = plsc.load_gather(rj, [rows, col2]) - plsc.load_gather(ri, [rows, col2])
            vb = (g >> 3) * 512 + (g & 7) * 16
            vec_v[pl.ds(vb, 16)] = v0
            vec_v[pl.ds(vb + 128, 16)] = v1
            vec_v[pl.ds(vb + 256, 16)] = v2
            d2 = v0 * v0 + v1 * v1 + v2 * v2
            d2c = jnp.maximum(d2, 1.1754944e-38)
            y = plsc.bitcast(0x5F3759DF - (plsc.bitcast(d2c, jnp.int32) >> 1),
                             jnp.float32)
            y = y * (1.5 - 0.5 * d2c * y * y)
            y = y * (1.5 - 0.5 * d2c * y * y)
            y = y * (1.5 - 0.5 * d2c * y * y)
            dist_v[pl.ds(o, 16)] = d2 * y
            return carry

        lax.fori_loop(0, BLKE // 16, group_body, 0, unroll=False)

    idx_descs = {0: issue_idx(0)}
    for d in idx_descs[0]:
        d.wait()
    g_descs = {0: issue_gathers(0)}
    idx_descs[1] = issue_idx(1)
    out_descs = {}
    for b in range(NBLK):
        if b + 1 < NBLK:
            for d in idx_descs[b + 1]:
                d.wait()
            g_descs[b + 1] = issue_gathers(b + 1)
        for d in g_descs[b]:
            d.wait()
        if b + 2 < NBLK:
            idx_descs[b + 2] = issue_idx(b + 2)
        if b - 2 in out_descs:
            for d in out_descs[b - 2]:
                d.wait()
        compute(b)
        out_descs[b] = issue_outs(b)
    for d in out_descs[NBLK - 2]:
        d.wait()
    for d in out_descs[NBLK - 1]:
        d.wait()

    @pl.when(wid < NTAIL)
    def _():
        qt = NW * CPW + wid
        et = qt * 128
        gi = idx[0][0].at[pl.ds(0, 128)]
        gj = idx[0][1].at[pl.ds(0, 128)]
        pltpu.sync_copy(bi_hbm.at[pl.ds(et, 128)], gi)
        pltpu.sync_copy(bj_hbm.at[pl.ds(et, 128)], gj)
        ri = gath[0][0].at[pl.ds(0, 128)]
        rj = gath[0][1].at[pl.ds(0, 128)]
        c1 = pltpu.async_copy(t8_hbm.at[gi], ri, sg0)
        c2 = pltpu.async_copy(t8_hbm.at[gj], rj, sg0)
        c1.wait()
        c2.wait()
        vec_v, dist_v = outs[0]

        def tail_group(g, carry):
            o = g * 16
            rows = iota16 + o
            u0 = plsc.load_gather(rj, [rows, col0]) - plsc.load_gather(ri, [rows, col0])
            u1 = plsc.load_gather(rj, [rows, col1]) - plsc.load_gather(ri, [rows, col1])
            u2 = plsc.load_gather(rj, [rows, col2]) - plsc.load_gather(ri, [rows, col2])
            vec_v[pl.ds(o, 16)] = u0
            vec_v[pl.ds(o + 128, 16)] = u1
            vec_v[pl.ds(o + 256, 16)] = u2
            d2 = u0 * u0 + u1 * u1 + u2 * u2
            d2c = jnp.maximum(d2, 1.1754944e-38)
            y = plsc.bitcast(0x5F3759DF - (plsc.bitcast(d2c, jnp.int32) >> 1),
                             jnp.float32)
            y = y * (1.5 - 0.5 * d2c * y * y)
            y = y * (1.5 - 0.5 * d2c * y * y)
            y = y * (1.5 - 0.5 * d2c * y * y)
            dist_v[pl.ds(o, 16)] = d2 * y
            return carry

        lax.fori_loop(0, 8, tail_group, 0, unroll=False)
        pltpu.sync_copy(vec_v.at[pl.ds(0, 512)],
                        ovec_hbm.at[pl.ds(qt * 512, 512)])
        pltpu.sync_copy(dist_v.at[pl.ds(0, 128)],
                        odist_hbm.at[pl.ds(et, 128)])


@jax.jit
def _sc_call(ax, ay, az, bi, bj):
    mesh = plsc.VectorSubcoreMesh(core_axis_name="c", subcore_axis_name="s",
                                  num_cores=NUM_CORES,
                                  num_subcores=NUM_SUBCORES)
    cparams = pltpu.CompilerParams(needs_layout_passes=False,
                                   use_tc_tiling_on_sc=False)
    build = pl.kernel(
        _build_body,
        out_type=jax.ShapeDtypeStruct((N_ATOMS, 8), jnp.float32),
        mesh=mesh,
        compiler_params=cparams,
        scratch_types=[
            pltpu.VMEM((APW,), jnp.float32),
            pltpu.VMEM((APW,), jnp.float32),
            pltpu.VMEM((APW,), jnp.float32),
            pltpu.VMEM((APW, 8), jnp.float32),
            pltpu.SemaphoreType.DMA,
        ],
    )
    t8 = build(ax, ay, az)
    ivec = pltpu.VMEM((BLKE,), jnp.int32)
    rvec = pltpu.VMEM((BLKE, 8), jnp.float32)
    vvec = pltpu.VMEM((BLKC * 512,), jnp.float32)
    dvec = pltpu.VMEM((BLKE,), jnp.float32)
    f = pl.kernel(
        _sc_body,
        out_type=(jax.ShapeDtypeStruct((NCHUNK * 512,), jnp.float32),
                  jax.ShapeDtypeStruct((N_EDGES,), jnp.float32)),
        mesh=mesh,
        compiler_params=cparams,
        scratch_types=(
            [ivec] * 4 + [rvec] * 4 + [vvec, dvec, vvec, dvec]
            + [pltpu.SemaphoreType.DMA] * 6
        ),
    )
    return f(t8, bi, bj)


def kernel(atoms_x, bonds_i, bonds_j):
    vec_raw, dist = _sc_call(atoms_x[:, 0], atoms_x[:, 1], atoms_x[:, 2],
                             bonds_i.astype(jnp.int32),
                             bonds_j.astype(jnp.int32))
    bond_vec = (vec_raw.reshape(NCHUNK, 4, 128)
                .transpose(0, 2, 1)
                .reshape(N_EDGES, 4)[:, :3])
    return bond_vec, dist
